# Initial kernel scaffold; baseline (speedup 1.0000x reference)
import functools

import jax
import jax.numpy as jnp
from jax import lax
from jax.experimental import pallas as pl
from jax.experimental.pallas import tpu as pltpu
from jax.experimental.pallas import tpu_sc as plsc

_E, _B, _S, _D = 8, 2, 2048, 1024
_K = 2
_PAD_E = 16
_S_BLK = 512
_NS = _S // _S_BLK
_C_BLK = 512
_NC = _S // _C_BLK
_NEG = -1e30


def _gate_body(x_ref, w_ref, bias_ref, gate_ref, acc_ref):
    e = pl.program_id(0)
    b = pl.program_id(1)
    s = pl.program_id(2)

    @pl.when((e == 0) & (b == 0) & (s == 0))
    def _():
        for bb in range(_B):
            for j in range(_E, _PAD_E):
                gate_ref[bb, j] = _NEG

    x = x_ref[0, 0]
    part = jnp.sum(
        lax.dot_general(x, w_ref[...], (((1,), (0,)), ((), ())),
                        preferred_element_type=jnp.float32))

    @pl.when(s == 0)
    def _():
        acc_ref[b, e] = part

    @pl.when(s != 0)
    def _():
        acc_ref[b, e] = acc_ref[b, e] + part

    @pl.when(s == _NS - 1)
    def _():
        gate_ref[b, e] = acc_ref[b, e] * (1.0 / _S) + bias_ref[0]


_gate_call = pl.pallas_call(
    _gate_body,
    grid=(_E, _B, _NS),
    in_specs=[
        pl.BlockSpec((1, 1, _S_BLK, _D), lambda e, b, s: (e, b, s, 0)),
        pl.BlockSpec((_D, 1), lambda e, b, s: (0, 0)),
        pl.BlockSpec(memory_space=pltpu.SMEM),
    ],
    out_specs=pl.BlockSpec(memory_space=pltpu.SMEM),
    out_shape=jax.ShapeDtypeStruct((_B, _PAD_E), jnp.float32),
    scratch_shapes=[pltpu.SMEM((_B, _E), jnp.float32)],
)


_sc_mesh = plsc.VectorSubcoreMesh(core_axis_name="c", subcore_axis_name="s")


@functools.partial(
    pl.kernel,
    out_type=(jax.ShapeDtypeStruct((_B, _PAD_E), jnp.int32),
              jax.ShapeDtypeStruct((_B, _PAD_E), jnp.float32)),
    mesh=_sc_mesh,
    scratch_types=[pltpu.VMEM((_B, _PAD_E), jnp.float32),
                   pltpu.VMEM((_B, _PAD_E), jnp.int32),
                   pltpu.VMEM((_B, _PAD_E), jnp.float32)],
)
def _route(gate_hbm, idx_out, score_out, gate_v, idx_v, score_v):
    cid = lax.axis_index("c")
    sid = lax.axis_index("s")

    @pl.when((cid == 0) & (sid == 0))
    def _():
        pltpu.sync_copy(gate_hbm, gate_v)
        lanes = lax.iota(jnp.int32, _PAD_E)
        for bi in range(_B):
            g = gate_v[bi]
            m1 = jnp.max(g, axis=0)
            i1 = plsc.all_reduce_ffs(g == m1)
            g2 = jnp.where(lanes == i1, _NEG, g)
            m2 = jnp.max(g2, axis=0)
            i2 = plsc.all_reduce_ffs(g2 == m2)
            ev = jnp.exp(jnp.broadcast_to(m2 - m1, (_PAD_E,)))
            p1 = 1.0 / (1.0 + ev)
            p2 = 1.0 - p1
            idx_v[bi] = jnp.where(lanes == 0, i1,
                                  jnp.where(lanes == 1, i2, 0))
            score_v[bi] = jnp.where(lanes == 0, p1,
                                    jnp.where(lanes == 1, p2, 0.0)) * (1.0 / _K)
        pltpu.sync_copy(idx_v, idx_out)
        pltpu.sync_copy(score_v, score_out)


def _combine_body(idx_ref, score_ref, x_ref, o_ref):
    b = pl.program_id(0)
    k = pl.program_id(2)
    w = score_ref[b * _K + k]
    contrib = x_ref[0, 0] * w

    @pl.when(k == 0)
    def _():
        o_ref[0] = contrib

    @pl.when(k != 0)
    def _():
        o_ref[0] = o_ref[0] + contrib


_combine_call = pl.pallas_call(
    _combine_body,
    grid_spec=pltpu.PrefetchScalarGridSpec(
        num_scalar_prefetch=2,
        grid=(_B, _NC, _K),
        in_specs=[
            pl.BlockSpec((1, 1, _C_BLK, _D),
                         lambda b, s, k, idx, scr: (idx[b * _K + k], b, s, 0)),
        ],
        out_specs=pl.BlockSpec((1, _C_BLK, _D),
                               lambda b, s, k, idx, scr: (b, s, 0)),
    ),
    out_shape=jax.ShapeDtypeStruct((_B, _S, _D), jnp.float32),
)


def kernel(hidden_states, W, b):
    gate = _gate_call(hidden_states, W, b)
    idx16, score16 = _route(gate)
    idx_flat = idx16[:, :_K].reshape(-1)
    score_flat = score16[:, :_K].reshape(-1)
    return _combine_call(idx_flat, score_flat, hidden_states)


# speedup vs baseline: 1.5640x; 1.5640x over previous
import functools

import jax
import jax.numpy as jnp
from jax import lax
from jax.experimental import pallas as pl
from jax.experimental.pallas import tpu as pltpu
from jax.experimental.pallas import tpu_sc as plsc

_E, _B, _S, _D = 8, 2, 2048, 1024
_K = 2
_PAD_E = 16
_S_BLK = 512
_NS = _S // _S_BLK
_C_BLK = 512
_NC = _S // _C_BLK
_NEG = -1e30


def _gate_body(x_ref, w_ref, bias_ref, gate_ref, acc_ref):
    e = pl.program_id(0)
    b = pl.program_id(1)
    s = pl.program_id(2)

    @pl.when((e == 0) & (b == 0) & (s == 0))
    def _():
        for ee in range(_E):
            for j in range(_B, _PAD_E):
                gate_ref[ee, j] = 0.0

    x = x_ref[0, 0]
    part = jnp.sum(
        lax.dot_general(x, w_ref[...], (((1,), (0,)), ((), ())),
                        preferred_element_type=jnp.float32))

    @pl.when(s == 0)
    def _():
        acc_ref[b, e] = part

    @pl.when(s != 0)
    def _():
        acc_ref[b, e] = acc_ref[b, e] + part

    @pl.when(s == _NS - 1)
    def _():
        gate_ref[e, b] = acc_ref[b, e] * (1.0 / _S) + bias_ref[0]


_gate_call = pl.pallas_call(
    _gate_body,
    grid=(_E, _B, _NS),
    in_specs=[
        pl.BlockSpec((1, 1, _S_BLK, _D), lambda e, b, s: (e, b, s, 0)),
        pl.BlockSpec((_D, 1), lambda e, b, s: (0, 0)),
        pl.BlockSpec(memory_space=pltpu.SMEM),
    ],
    out_specs=pl.BlockSpec(memory_space=pltpu.SMEM),
    out_shape=jax.ShapeDtypeStruct((_E, _PAD_E), jnp.float32),
    scratch_shapes=[pltpu.SMEM((_B, _E), jnp.float32)],
)


def _route_body(gate_hbm, idx_out, score_out, gate_v, idx_v, score_v):
    cid = lax.axis_index("c")
    sid = lax.axis_index("s")

    @pl.when((cid == 0) & (sid == 0))
    def _():
        pltpu.sync_copy(gate_hbm, gate_v)
        g = [gate_v[e] for e in range(_E)]
        zeros_i = jnp.zeros((_PAD_E,), jnp.int32)
        best_v = g[0]
        best_i = zeros_i
        for e in range(1, _E):
            better = g[e] > best_v
            best_i = jnp.where(better, zeros_i + e, best_i)
            best_v = jnp.where(better, g[e], best_v)
        sec_v = jnp.full((_PAD_E,), _NEG, jnp.float32)
        sec_i = zeros_i
        for e in range(_E):
            better = (g[e] > sec_v) & (best_i != e)
            sec_i = jnp.where(better, zeros_i + e, sec_i)
            sec_v = jnp.where(better, g[e], sec_v)
        ev = jnp.exp(sec_v - best_v)
        p1 = 1.0 / (1.0 + ev)
        idx_v[0] = best_i
        idx_v[1] = sec_i
        score_v[0] = p1 * (1.0 / _K)
        score_v[1] = (1.0 - p1) * (1.0 / _K)
        pltpu.sync_copy(idx_v, idx_out)
        pltpu.sync_copy(score_v, score_out)


_route_call = None


def _get_route():
    global _route_call
    if _route_call is None:
        _route_call = pl.kernel(
            _route_body,
            out_type=(jax.ShapeDtypeStruct((_K, _PAD_E), jnp.int32),
                      jax.ShapeDtypeStruct((_K, _PAD_E), jnp.float32)),
            mesh=plsc.VectorSubcoreMesh(core_axis_name="c",
                                        subcore_axis_name="s"),
            scratch_types=[pltpu.VMEM((_E, _PAD_E), jnp.float32),
                           pltpu.VMEM((_K, _PAD_E), jnp.int32),
                           pltpu.VMEM((_K, _PAD_E), jnp.float32)],
        )
    return _route_call


def _combine_body(idx_ref, score_ref, x_ref, o_ref):
    b = pl.program_id(0)
    k = pl.program_id(2)
    w = score_ref[b * _K + k]
    contrib = x_ref[0, 0] * w

    @pl.when(k == 0)
    def _():
        o_ref[0] = contrib

    @pl.when(k != 0)
    def _():
        o_ref[0] = o_ref[0] + contrib


_combine_call = pl.pallas_call(
    _combine_body,
    grid_spec=pltpu.PrefetchScalarGridSpec(
        num_scalar_prefetch=2,
        grid=(_B, _NC, _K),
        in_specs=[
            pl.BlockSpec((1, 1, _C_BLK, _D),
                         lambda b, s, k, idx, scr: (idx[b * _K + k], b, s, 0)),
        ],
        out_specs=pl.BlockSpec((1, _C_BLK, _D),
                               lambda b, s, k, idx, scr: (b, s, 0)),
    ),
    out_shape=jax.ShapeDtypeStruct((_B, _S, _D), jnp.float32),
)


def kernel(hidden_states, W, b):
    gate_t = _gate_call(hidden_states, W, b)
    idx2, score2 = _get_route()(gate_t)
    idx_flat = idx2[:, :_B].T.reshape(-1)
    score_flat = score2[:, :_B].T.reshape(-1)
    return _combine_call(idx_flat, score_flat, hidden_states)


# speedup vs baseline: 2.2654x; 1.4484x over previous
import functools

import jax
import jax.numpy as jnp
from jax import lax
from jax.experimental import pallas as pl
from jax.experimental.pallas import tpu as pltpu
from jax.experimental.pallas import tpu_sc as plsc

_E, _B, _S, _D = 8, 2, 2048, 1024
_K = 2
_PAD_E = 16
_C_BLK = 1024
_NC = _S // _C_BLK
_NEG = -1e30


def _gate_body(x_ref, w_ref, bias_ref, gate_ref):
    e = pl.program_id(0)
    b = pl.program_id(1)

    @pl.when((e == 0) & (b == 0))
    def _():
        for ee in range(_E):
            for j in range(_B, _PAD_E):
                gate_ref[ee, j] = 0.0

    x = x_ref[0, 0]
    total = jnp.sum(
        lax.dot_general(x, w_ref[...], (((1,), (0,)), ((), ())),
                        preferred_element_type=jnp.float32))
    gate_ref[e, b] = total * (1.0 / _S) + bias_ref[0]


_gate_call = pl.pallas_call(
    _gate_body,
    grid=(_E, _B),
    in_specs=[
        pl.BlockSpec((1, 1, _S, _D), lambda e, b: (e, b, 0, 0)),
        pl.BlockSpec((_D, 1), lambda e, b: (0, 0)),
        pl.BlockSpec(memory_space=pltpu.SMEM),
    ],
    out_specs=pl.BlockSpec(memory_space=pltpu.SMEM),
    out_shape=jax.ShapeDtypeStruct((_E, _PAD_E), jnp.float32),
)


def _route_body(gate_hbm, idx_out, score_out, gate_v, idx_v, score_v):
    cid = lax.axis_index("c")
    sid = lax.axis_index("s")

    @pl.when((cid == 0) & (sid == 0))
    def _():
        pltpu.sync_copy(gate_hbm, gate_v)
        g = [gate_v[e] for e in range(_E)]
        zeros_i = jnp.zeros((_PAD_E,), jnp.int32)
        best_v = g[0]
        best_i = zeros_i
        for e in range(1, _E):
            better = g[e] > best_v
            best_i = jnp.where(better, zeros_i + e, best_i)
            best_v = jnp.where(better, g[e], best_v)
        sec_v = jnp.full((_PAD_E,), _NEG, jnp.float32)
        sec_i = zeros_i
        for e in range(_E):
            better = (g[e] > sec_v) & (best_i != e)
            sec_i = jnp.where(better, zeros_i + e, sec_i)
            sec_v = jnp.where(better, g[e], sec_v)
        ev = jnp.exp(sec_v - best_v)
        p1 = 1.0 / (1.0 + ev)
        idx_v[0] = best_i
        idx_v[1] = sec_i
        score_v[0] = p1 * (1.0 / _K)
        score_v[1] = (1.0 - p1) * (1.0 / _K)
        pltpu.sync_copy(idx_v, idx_out)
        pltpu.sync_copy(score_v, score_out)


_route_call = None


def _get_route():
    global _route_call
    if _route_call is None:
        _route_call = pl.kernel(
            _route_body,
            out_type=(jax.ShapeDtypeStruct((_K, _PAD_E), jnp.int32),
                      jax.ShapeDtypeStruct((_K, _PAD_E), jnp.float32)),
            mesh=plsc.VectorSubcoreMesh(core_axis_name="c",
                                        subcore_axis_name="s"),
            scratch_types=[pltpu.VMEM((_E, _PAD_E), jnp.float32),
                           pltpu.VMEM((_K, _PAD_E), jnp.int32),
                           pltpu.VMEM((_K, _PAD_E), jnp.float32)],
        )
    return _route_call


def _combine_body(idx_ref, score_ref, x0_ref, x1_ref, o_ref):
    b = pl.program_id(0)
    w0 = score_ref[b * _K]
    w1 = score_ref[b * _K + 1]
    o_ref[0] = x0_ref[0, 0] * w0 + x1_ref[0, 0] * w1


_combine_call = pl.pallas_call(
    _combine_body,
    grid_spec=pltpu.PrefetchScalarGridSpec(
        num_scalar_prefetch=2,
        grid=(_B, _NC),
        in_specs=[
            pl.BlockSpec((1, 1, _C_BLK, _D),
                         lambda b, s, idx, scr: (idx[b * _K], b, s, 0)),
            pl.BlockSpec((1, 1, _C_BLK, _D),
                         lambda b, s, idx, scr: (idx[b * _K + 1], b, s, 0)),
        ],
        out_specs=pl.BlockSpec((1, _C_BLK, _D),
                               lambda b, s, idx, scr: (b, s, 0)),
    ),
    out_shape=jax.ShapeDtypeStruct((_B, _S, _D), jnp.float32),
)


def kernel(hidden_states, W, b):
    gate_t = _gate_call(hidden_states, W, b)
    idx2, score2 = _get_route()(gate_t)
    idx_flat = idx2[:, :_B].T.reshape(-1)
    score_flat = score2[:, :_B].T.reshape(-1)
    return _combine_call(idx_flat, score_flat, hidden_states, hidden_states)


# speedup vs baseline: 2.3076x; 1.0187x over previous
import functools

import jax
import jax.numpy as jnp
from jax import lax
from jax.experimental import pallas as pl
from jax.experimental.pallas import tpu as pltpu
from jax.experimental.pallas import tpu_sc as plsc

_E, _B, _S, _D = 8, 2, 2048, 1024
_K = 2
_PAD_E = 16
_C_BLK = 1024
_NC = _S // _C_BLK
_NEG = -1e30


def _gate_body(x_ref, w_ref, bias_ref, gate_ref):
    e = pl.program_id(0)

    @pl.when(e == 0)
    def _():
        for ee in range(_E):
            for j in range(_B, _PAD_E):
                gate_ref[ee, j] = 0.0

    x = x_ref[0].reshape(_B * _S, _D)
    y = lax.dot_general(x, w_ref[...], (((1,), (0,)), ((), ())),
                        preferred_element_type=jnp.float32)
    for b in range(_B):
        gate_ref[e, b] = (jnp.sum(y[b * _S:(b + 1) * _S]) * (1.0 / _S)
                          + bias_ref[0])


_gate_call = pl.pallas_call(
    _gate_body,
    grid=(_E,),
    in_specs=[
        pl.BlockSpec((1, _B, _S, _D), lambda e: (e, 0, 0, 0)),
        pl.BlockSpec((_D, 1), lambda e: (0, 0)),
        pl.BlockSpec(memory_space=pltpu.SMEM),
    ],
    out_specs=pl.BlockSpec(memory_space=pltpu.SMEM),
    out_shape=jax.ShapeDtypeStruct((_E, _PAD_E), jnp.float32),
)


def _route_body(gate_hbm, idx_out, score_out, gate_v, idx_v, score_v):
    cid = lax.axis_index("c")
    sid = lax.axis_index("s")

    @pl.when((cid == 0) & (sid == 0))
    def _():
        pltpu.sync_copy(gate_hbm, gate_v)
        g = [gate_v[e] for e in range(_E)]
        zeros_i = jnp.zeros((_PAD_E,), jnp.int32)
        best_v = g[0]
        best_i = zeros_i
        for e in range(1, _E):
            better = g[e] > best_v
            best_i = jnp.where(better, zeros_i + e, best_i)
            best_v = jnp.where(better, g[e], best_v)
        sec_v = jnp.full((_PAD_E,), _NEG, jnp.float32)
        sec_i = zeros_i
        for e in range(_E):
            better = (g[e] > sec_v) & (best_i != e)
            sec_i = jnp.where(better, zeros_i + e, sec_i)
            sec_v = jnp.where(better, g[e], sec_v)
        ev = jnp.exp(sec_v - best_v)
        p1 = 1.0 / (1.0 + ev)
        idx_v[0] = best_i
        idx_v[1] = sec_i
        score_v[0] = p1 * (1.0 / _K)
        score_v[1] = (1.0 - p1) * (1.0 / _K)
        pltpu.sync_copy(idx_v, idx_out)
        pltpu.sync_copy(score_v, score_out)


_route_call = None


def _get_route():
    global _route_call
    if _route_call is None:
        _route_call = pl.kernel(
            _route_body,
            out_type=(jax.ShapeDtypeStruct((_K, _PAD_E), jnp.int32),
                      jax.ShapeDtypeStruct((_K, _PAD_E), jnp.float32)),
            mesh=plsc.VectorSubcoreMesh(core_axis_name="c",
                                        subcore_axis_name="s"),
            scratch_types=[pltpu.VMEM((_E, _PAD_E), jnp.float32),
                           pltpu.VMEM((_K, _PAD_E), jnp.int32),
                           pltpu.VMEM((_K, _PAD_E), jnp.float32)],
        )
    return _route_call


def _combine_body(idx_ref, score_ref, x0_ref, x1_ref, o_ref):
    b = pl.program_id(0)
    w0 = score_ref[0, b]
    w1 = score_ref[1, b]
    o_ref[0] = x0_ref[0, 0] * w0 + x1_ref[0, 0] * w1


_combine_call = pl.pallas_call(
    _combine_body,
    grid_spec=pltpu.PrefetchScalarGridSpec(
        num_scalar_prefetch=2,
        grid=(_B, _NC),
        in_specs=[
            pl.BlockSpec((1, 1, _C_BLK, _D),
                         lambda b, s, idx, scr: (idx[0, b], b, s, 0)),
            pl.BlockSpec((1, 1, _C_BLK, _D),
                         lambda b, s, idx, scr: (idx[1, b], b, s, 0)),
        ],
        out_specs=pl.BlockSpec((1, _C_BLK, _D),
                               lambda b, s, idx, scr: (b, s, 0)),
    ),
    out_shape=jax.ShapeDtypeStruct((_B, _S, _D), jnp.float32),
)


def kernel(hidden_states, W, b):
    gate_t = _gate_call(hidden_states, W, b)
    idx2, score2 = _get_route()(gate_t)
    return _combine_call(idx2, score2, hidden_states, hidden_states)
